# Initial kernel scaffold; baseline (speedup 1.0000x reference)
import jax
import jax.numpy as jnp
from jax.experimental import pallas as pl


def kernel(x, df, d2f):
    raise NotImplementedError("write your pallas kernel here")


# speedup vs baseline: 2.0296x; 2.0296x over previous
import functools

import jax
import jax.numpy as jnp
import numpy as np
from jax.experimental import pallas as pl

_EPS = 0.001
_EPS2 = _EPS * _EPS
_DIL = 3.0
_TMAX = 3.0
_TMIN = float(np.log(0.075))
_LOGEPS = float(np.log(_EPS))


def _op_kernel(x_ref, df_ref, d2f_ref, h_ref, la_ref, phi_ref, lt_ref, psi_ref):
    x = x_ref[0]

    xl = jnp.roll(x, -1, axis=2)
    xr = jnp.roll(x, 1, axis=2)
    xu = jnp.roll(x, -1, axis=1)
    xd = jnp.roll(x, 1, axis=1)

    dx = 0.5 * (xl - xr)
    dy = 0.5 * (xu - xd)
    dxx = xl + xr - 2.0 * x
    dyy = xu + xd - 2.0 * x

    df0 = df_ref[0, :, 0:1][:, :, None]
    df1 = df_ref[0, :, 1:2][:, :, None]
    d2fr = d2f_ref[0, :, 0:1][:, :, None]
    d2fi = d2f_ref[0, :, 1:2][:, :, None]
    edf = jnp.exp(df0)
    dfr = edf * jnp.cos(df1)
    dfi = edf * jnp.sin(df1)
    inv_mag2 = 1.0 / (edf * edf)
    t2r = (d2fr * dfr + d2fi * dfi) * inv_mag2
    t2i = (d2fi * dfr - d2fr * dfi) * inv_mag2

    h = dx * dx + dy * dy
    mask = h >= _EPS2
    dxs_r = jnp.where(mask, dx, 1.0)
    dxs_i = jnp.where(mask, dy, 0.0)
    hs = jnp.where(mask, h, 1.0)

    ln_alpha = jnp.clip(df0 - 0.5 * jnp.log(hs), -_DIL, _DIL)
    la_ref[0] = jnp.where(mask, ln_alpha, 0.0)

    phi = jnp.arctan2(dfi * dxs_r - dfr * dxs_i, dfr * dxs_r + dfi * dxs_i)
    phi_ref[0] = jnp.where(mask, phi, 0.0)

    nr = dxx * dfr - dyy * dfi
    ni = dxx * dfi + dyy * dfr
    wr = dxs_r * dxs_r - dxs_i * dxs_i
    wi = 2.0 * dxs_r * dxs_i
    inv_hs2 = 1.0 / (hs * hs)
    qr = (nr * wr + ni * wi) * inv_hs2
    qi = (ni * wr - nr * wi) * inv_hs2
    anr = 0.5 * (qr - t2r)
    ani = 0.5 * (qi - t2i)

    han = anr * anr + ani * ani
    mask_t = jnp.logical_and(mask, han >= _EPS2)
    ans_r = jnp.where(mask_t, anr, 1.0)
    ans_i = jnp.where(mask_t, ani, 0.0)
    hans = jnp.where(mask_t, han, 1.0)

    ln_tau = jnp.clip(0.5 * jnp.log(hans), _TMIN, _TMAX)
    lt_ref[0] = jnp.where(mask_t, ln_tau, _LOGEPS)
    psi = jnp.arctan2(ans_i, ans_r)
    psi_ref[0] = jnp.where(mask_t, psi, 0.0)

    h_ref[0] = h


@functools.partial(jax.jit, static_argnames=("interpret",))
def kernel(x, df, d2f, interpret=False):
    b, C, H, W = x.shape
    CB = 32
    grid = (b, C // CB)
    img_spec = pl.BlockSpec((1, CB, H, W), lambda i, j: (i, j, 0, 0))
    par_spec = pl.BlockSpec((1, CB, 2), lambda i, j: (0, j, 0))
    out = pl.pallas_call(
        _op_kernel,
        grid=grid,
        in_specs=[img_spec, par_spec, par_spec],
        out_specs=[img_spec] * 5,
        out_shape=[jax.ShapeDtypeStruct((b, C, H, W), jnp.float32)] * 5,
        interpret=interpret,
    )(x, df, d2f)
    return tuple(out)


# speedup vs baseline: 2.5140x; 1.2387x over previous
import functools

import jax
import jax.numpy as jnp
import numpy as np
from jax.experimental import pallas as pl

_EPS = 0.001
_EPS2 = _EPS * _EPS
_DIL = 3.0
_TMAX = 3.0
_TMIN = float(np.log(0.075))
_LOGEPS = float(np.log(_EPS))


def _fast_atan2(y, x):
    ax = jnp.abs(x)
    ay = jnp.abs(y)
    mx = jnp.maximum(ax, ay)
    mn = jnp.minimum(ax, ay)
    a = mn / jnp.maximum(mx, 1e-30)
    s = a * a
    r = a * (0.99997726 + s * (-0.33262347 + s * (0.19354346 + s * (
        -0.11643287 + s * (0.05265332 - s * 0.01172120)))))
    r = jnp.where(ay > ax, 1.57079637 - r, r)
    r = jnp.where(x < 0, 3.14159274 - r, r)
    return jnp.where(y < 0, -r, r)


def _op_kernel(x_ref, df_ref, d2f_ref, h_ref, la_ref, phi_ref, lt_ref, psi_ref):
    x = x_ref[0]

    xl = jnp.roll(x, -1, axis=2)
    xr = jnp.roll(x, 1, axis=2)
    xu = jnp.roll(x, -1, axis=1)
    xd = jnp.roll(x, 1, axis=1)

    dx = 0.5 * (xl - xr)
    dy = 0.5 * (xu - xd)
    dxx = xl + xr - 2.0 * x
    dyy = xu + xd - 2.0 * x

    df0 = df_ref[0, :, 0:1][:, :, None]
    df1 = df_ref[0, :, 1:2][:, :, None]
    d2fr = d2f_ref[0, :, 0:1][:, :, None]
    d2fi = d2f_ref[0, :, 1:2][:, :, None]
    edf = jnp.exp(df0)
    dfr = edf * jnp.cos(df1)
    dfi = edf * jnp.sin(df1)
    inv_mag2 = 1.0 / (edf * edf)
    t2r = (d2fr * dfr + d2fi * dfi) * inv_mag2
    t2i = (d2fi * dfr - d2fr * dfi) * inv_mag2

    h = dx * dx + dy * dy
    mask = h >= _EPS2
    dxs_r = jnp.where(mask, dx, 1.0)
    dxs_i = jnp.where(mask, dy, 0.0)
    hs = jnp.where(mask, h, 1.0)

    ln_alpha = jnp.clip(df0 - 0.5 * jnp.log(hs), -_DIL, _DIL)
    la_ref[0] = jnp.where(mask, ln_alpha, 0.0)

    phi = _fast_atan2(dfi * dxs_r - dfr * dxs_i, dfr * dxs_r + dfi * dxs_i)
    phi_ref[0] = jnp.where(mask, phi, 0.0)

    nr = dxx * dfr - dyy * dfi
    ni = dxx * dfi + dyy * dfr
    wr = dxs_r * dxs_r - dxs_i * dxs_i
    wi = 2.0 * dxs_r * dxs_i
    inv_hs2 = 1.0 / (hs * hs)
    qr = (nr * wr + ni * wi) * inv_hs2
    qi = (ni * wr - nr * wi) * inv_hs2
    anr = 0.5 * (qr - t2r)
    ani = 0.5 * (qi - t2i)

    han = anr * anr + ani * ani
    mask_t = jnp.logical_and(mask, han >= _EPS2)
    ans_r = jnp.where(mask_t, anr, 1.0)
    ans_i = jnp.where(mask_t, ani, 0.0)
    hans = jnp.where(mask_t, han, 1.0)

    ln_tau = jnp.clip(0.5 * jnp.log(hans), _TMIN, _TMAX)
    lt_ref[0] = jnp.where(mask_t, ln_tau, _LOGEPS)
    psi = _fast_atan2(ans_i, ans_r)
    psi_ref[0] = jnp.where(mask_t, psi, 0.0)

    h_ref[0] = h


@functools.partial(jax.jit, static_argnames=("interpret",))
def kernel(x, df, d2f, interpret=False):
    b, C, H, W = x.shape
    CB = 32
    grid = (b, C // CB)
    img_spec = pl.BlockSpec((1, CB, H, W), lambda i, j: (i, j, 0, 0))
    par_spec = pl.BlockSpec((1, CB, 2), lambda i, j: (0, j, 0))
    out = pl.pallas_call(
        _op_kernel,
        grid=grid,
        in_specs=[img_spec, par_spec, par_spec],
        out_specs=[img_spec] * 5,
        out_shape=[jax.ShapeDtypeStruct((b, C, H, W), jnp.float32)] * 5,
        interpret=interpret,
    )(x, df, d2f)
    return tuple(out)


# speedup vs baseline: 2.5911x; 1.0306x over previous
import functools

import jax
import jax.numpy as jnp
import numpy as np
from jax.experimental import pallas as pl

_EPS = 0.001
_EPS2 = _EPS * _EPS
_DIL = 3.0
_TMAX = 3.0
_TMIN = float(np.log(0.075))
_LOGEPS = float(np.log(_EPS))


def _fast_atan2(y, x):
    ax = jnp.abs(x)
    ay = jnp.abs(y)
    mx = jnp.maximum(ax, ay)
    mn = jnp.minimum(ax, ay)
    a = mn / jnp.maximum(mx, 1e-30)
    s = a * a
    r = a * (0.99997726 + s * (-0.33262347 + s * (0.19354346 + s * (
        -0.11643287 + s * (0.05265332 - s * 0.01172120)))))
    r = jnp.where(ay > ax, 1.57079637 - r, r)
    r = jnp.where(x < 0, 3.14159274 - r, r)
    return jnp.where(y < 0, -r, r)


def _op_kernel(x_ref, df_ref, d2f_ref, h_ref, la_ref, phi_ref, lt_ref, psi_ref):
    x = x_ref[0]

    xl = jnp.roll(x, -1, axis=2)
    xr = jnp.roll(x, 1, axis=2)
    xu = jnp.roll(x, -1, axis=1)
    xd = jnp.roll(x, 1, axis=1)

    dx = 0.5 * (xl - xr)
    dy = 0.5 * (xu - xd)
    dxx = xl + xr - 2.0 * x
    dyy = xu + xd - 2.0 * x

    df0 = df_ref[0, :, 0:1][:, :, None]
    df1 = df_ref[0, :, 1:2][:, :, None]
    d2fr = d2f_ref[0, :, 0:1][:, :, None]
    d2fi = d2f_ref[0, :, 1:2][:, :, None]
    edf = jnp.exp(df0)
    dfr = edf * jnp.cos(df1)
    dfi = edf * jnp.sin(df1)
    inv_mag2 = 1.0 / (edf * edf)
    t2r = (d2fr * dfr + d2fi * dfi) * inv_mag2
    t2i = (d2fi * dfr - d2fr * dfi) * inv_mag2

    h = dx * dx + dy * dy
    mask = h >= _EPS2

    ln_alpha = jnp.clip(df0 - 0.5 * jnp.log(h), -_DIL, _DIL)
    la_ref[0] = jnp.where(mask, ln_alpha, 0.0)

    phi = _fast_atan2(dfi * dx - dfr * dy, dfr * dx + dfi * dy)
    phi_ref[0] = jnp.where(mask, phi, 0.0)

    nr = dxx * dfr - dyy * dfi
    ni = dxx * dfi + dyy * dfr
    wr = dx * dx - dy * dy
    wi = 2.0 * dx * dy
    inv_h2 = 1.0 / (h * h)
    qr = (nr * wr + ni * wi) * inv_h2
    qi = (ni * wr - nr * wi) * inv_h2
    anr = 0.5 * (qr - t2r)
    ani = 0.5 * (qi - t2i)

    han = anr * anr + ani * ani
    mask_t = jnp.logical_and(mask, han >= _EPS2)
    ans_r = jnp.where(mask_t, anr, 1.0)
    ans_i = jnp.where(mask_t, ani, 0.0)

    ln_tau = jnp.clip(0.5 * jnp.log(han), _TMIN, _TMAX)
    lt_ref[0] = jnp.where(mask_t, ln_tau, _LOGEPS)
    psi = _fast_atan2(ans_i, ans_r)
    psi_ref[0] = jnp.where(mask_t, psi, 0.0)

    h_ref[0] = h


@functools.partial(jax.jit, static_argnames=("interpret",))
def kernel(x, df, d2f, interpret=False):
    b, C, H, W = x.shape
    CB = 32
    grid = (b, C // CB)
    img_spec = pl.BlockSpec((1, CB, H, W), lambda i, j: (i, j, 0, 0))
    par_spec = pl.BlockSpec((1, CB, 2), lambda i, j: (0, j, 0))
    out = pl.pallas_call(
        _op_kernel,
        grid=grid,
        in_specs=[img_spec, par_spec, par_spec],
        out_specs=[img_spec] * 5,
        out_shape=[jax.ShapeDtypeStruct((b, C, H, W), jnp.float32)] * 5,
        interpret=interpret,
    )(x, df, d2f)
    return tuple(out)


# speedup vs baseline: 4.1499x; 1.6016x over previous
import functools

import jax
import jax.numpy as jnp
import numpy as np
from jax.experimental import pallas as pl

_EPS = 0.001
_EPS2 = _EPS * _EPS
_DIL = 3.0
_TMAX = 3.0
_TMIN = float(np.log(0.075))
_LOGEPS = float(np.log(_EPS))


def _fast_atan2(y, x):
    ax = jnp.abs(x)
    ay = jnp.abs(y)
    mx = jnp.maximum(ax, ay)
    mn = jnp.minimum(ax, ay)
    a = mn / jnp.maximum(mx, 1e-30)
    s = a * a
    r = a * (0.99997726 + s * (-0.33262347 + s * (0.19354346 + s * (
        -0.11643287 + s * (0.05265332 - s * 0.01172120)))))
    r = jnp.where(ay > ax, 1.57079637 - r, r)
    r = jnp.where(x < 0, 3.14159274 - r, r)
    return jnp.where(y < 0, -r, r)


def _op_kernel(x_ref, df_ref, d2f_ref, h_ref, la_ref, phi_ref, lt_ref, psi_ref):
    cb = x_ref.shape[1]

    df0c = df_ref[0, :, 0:1]
    df1c = df_ref[0, :, 1:2]
    d2frc = d2f_ref[0, :, 0:1]
    d2fic = d2f_ref[0, :, 1:2]
    edfc = jnp.exp(df0c)
    dfrc = edfc * jnp.cos(df1c)
    dfic = edfc * jnp.sin(df1c)
    inv_mag2 = 1.0 / (edfc * edfc)
    t2rc = (d2frc * dfrc + d2fic * dfic) * inv_mag2
    t2ic = (d2fic * dfrc - d2frc * dfic) * inv_mag2

    for c in range(cb):
        _channel(x_ref, h_ref, la_ref, phi_ref, lt_ref, psi_ref, c,
                 df0c[c, 0], dfrc[c, 0], dfic[c, 0], t2rc[c, 0], t2ic[c, 0])


def _channel(x_ref, h_ref, la_ref, phi_ref, lt_ref, psi_ref, c,
             df0, dfr, dfi, t2r, t2i):
    x = x_ref[0, c]

    xl = jnp.roll(x, -1, axis=1)
    xr = jnp.roll(x, 1, axis=1)
    xu = jnp.roll(x, -1, axis=0)
    xd = jnp.roll(x, 1, axis=0)

    dx = 0.5 * (xl - xr)
    dy = 0.5 * (xu - xd)
    dxx = xl + xr - 2.0 * x
    dyy = xu + xd - 2.0 * x

    h = dx * dx + dy * dy
    mask = h >= _EPS2

    ln_alpha = jnp.clip(df0 - 0.5 * jnp.log(h), -_DIL, _DIL)
    la_ref[0, c] = jnp.where(mask, ln_alpha, 0.0)

    phi = _fast_atan2(dfi * dx - dfr * dy, dfr * dx + dfi * dy)
    phi_ref[0, c] = jnp.where(mask, phi, 0.0)

    nr = dxx * dfr - dyy * dfi
    ni = dxx * dfi + dyy * dfr
    wr = dx * dx - dy * dy
    wi = 2.0 * dx * dy
    inv_h2 = 1.0 / (h * h)
    qr = (nr * wr + ni * wi) * inv_h2
    qi = (ni * wr - nr * wi) * inv_h2
    anr = 0.5 * (qr - t2r)
    ani = 0.5 * (qi - t2i)

    han = anr * anr + ani * ani
    mask_t = jnp.logical_and(mask, han >= _EPS2)
    ans_r = jnp.where(mask_t, anr, 1.0)
    ans_i = jnp.where(mask_t, ani, 0.0)

    ln_tau = jnp.clip(0.5 * jnp.log(han), _TMIN, _TMAX)
    lt_ref[0, c] = jnp.where(mask_t, ln_tau, _LOGEPS)
    psi = _fast_atan2(ans_i, ans_r)
    psi_ref[0, c] = jnp.where(mask_t, psi, 0.0)

    h_ref[0, c] = h


@functools.partial(jax.jit, static_argnames=("interpret",))
def kernel(x, df, d2f, interpret=False):
    b, C, H, W = x.shape
    CB = 32
    grid = (b, C // CB)
    img_spec = pl.BlockSpec((1, CB, H, W), lambda i, j: (i, j, 0, 0))
    par_spec = pl.BlockSpec((1, CB, 2), lambda i, j: (0, j, 0))
    out = pl.pallas_call(
        _op_kernel,
        grid=grid,
        in_specs=[img_spec, par_spec, par_spec],
        out_specs=[img_spec] * 5,
        out_shape=[jax.ShapeDtypeStruct((b, C, H, W), jnp.float32)] * 5,
        interpret=interpret,
    )(x, df, d2f)
    return tuple(out)
